# Initial kernel scaffold; baseline (speedup 1.0000x reference)
import jax
import jax.numpy as jnp
from jax.experimental import pallas as pl


def kernel(objectness, box_regression, anchors):
    raise NotImplementedError("write your pallas kernel here")


# speedup vs baseline: 4.8100x; 4.8100x over previous
import functools
import math

import jax
import jax.numpy as jnp
from jax.experimental import pallas as pl
from jax.experimental.pallas import tpu as pltpu

_PRE = 5000
_POST = 1000
_THRESH = 0.7
_CLIP = math.log(1000.0 / 16.0)


def _rpn_kernel(num, R, logits_ref, breg_ref, anc_ref, out_ref,
                s_p, s_x1, s_y1, s_x2, s_y2, s_ar):
    TOT = R * 128
    p = jax.nn.sigmoid(logits_ref[0])
    s_p[...] = p

    row_i = jax.lax.broadcasted_iota(jnp.int32, (R, 128), 0)
    col_i = jax.lax.broadcasted_iota(jnp.int32, (R, 128), 1)
    flat = row_i * 128 + col_i
    valid = flat < num

    kbits = jax.lax.bitcast_convert_type(p, jnp.int32)
    pre = min(_PRE, num)

    def bis_body(_, carry):
        lo, hi = carry
        done = (hi - lo) <= 1
        mid = lo + (hi - lo) // 2
        cnt = jnp.sum(jnp.where(valid & (kbits >= mid), 1, 0))
        lo_n = jnp.where(cnt >= pre, mid, lo)
        hi_n = jnp.where(cnt >= pre, hi, mid)
        return (jnp.where(done, lo, lo_n), jnp.where(done, hi, hi_n))

    T, _ = jax.lax.fori_loop(
        0, 31, bis_body, (jnp.int32(-1), jnp.int32(0x7F800000)))

    n_gt = jnp.sum(jnp.where(valid & (kbits > T), 1, 0))
    need = pre - n_gt
    eq = valid & (kbits == T)

    idx_iters = max(1, math.ceil(math.log2(TOT)))

    def ibis_body(_, carry):
        lo, hi = carry
        done = (hi - lo) <= 1
        mid = lo + (hi - lo) // 2
        cnt = jnp.sum(jnp.where(eq & (flat < mid), 1, 0))
        lo_n = jnp.where(cnt >= need, lo, mid)
        hi_n = jnp.where(cnt >= need, mid, hi)
        return (jnp.where(done, lo, lo_n), jnp.where(done, hi, hi_n))

    _, cutoff = jax.lax.fori_loop(
        0, idx_iters, ibis_body, (jnp.int32(0), jnp.int32(TOT)))

    eligible = valid & ((kbits > T) | (eq & (flat < cutoff)))

    dx = breg_ref[0, 0]
    dy = breg_ref[0, 1]
    dw = breg_ref[0, 2]
    dh = breg_ref[0, 3]
    ax1 = anc_ref[0, 0]
    ay1 = anc_ref[0, 1]
    ax2 = anc_ref[0, 2]
    ay2 = anc_ref[0, 3]
    w = ax2 - ax1 + 1.0
    h = ay2 - ay1 + 1.0
    cx = ax1 + 0.5 * w
    cy = ay1 + 0.5 * h
    pcx = dx * w + cx
    pcy = dy * h + cy
    pw = jnp.exp(jnp.minimum(dw, _CLIP)) * w
    ph = jnp.exp(jnp.minimum(dh, _CLIP)) * h
    x1 = pcx - 0.5 * pw
    y1 = pcy - 0.5 * ph
    x2 = pcx + 0.5 * pw - 1.0
    y2 = pcy + 0.5 * ph - 1.0
    s_x1[...] = x1
    s_y1[...] = y1
    s_x2[...] = x2
    s_y2[...] = y2
    s_ar[...] = (x2 - x1 + 1.0) * (y2 - y1 + 1.0)

    m0 = jnp.where(eligible, p, -1.0)
    lane = jax.lax.broadcasted_iota(jnp.int32, (1, 128), 1)

    def nms_body(i, m):
        gmax = jnp.max(m)
        idx = jnp.min(jnp.where(m == gmax, flat, TOT))
        vp = gmax > -0.5
        r = idx // 128
        c = idx % 128

        def pick(ref):
            row = ref[pl.ds(r, 1), :]
            return jnp.sum(jnp.where(lane == c, row, 0.0))

        bx1 = pick(s_x1)
        by1 = pick(s_y1)
        bx2 = pick(s_x2)
        by2 = pick(s_y2)
        bar = pick(s_ar)

        ltx = jnp.maximum(bx1, s_x1[...])
        lty = jnp.maximum(by1, s_y1[...])
        rbx = jnp.minimum(bx2, s_x2[...])
        rby = jnp.minimum(by2, s_y2[...])
        iw = jnp.maximum(rbx - ltx + 1.0, 0.0)
        ih = jnp.maximum(rby - lty + 1.0, 0.0)
        inter = iw * ih
        iou = inter / (bar + s_ar[...] - inter)
        m_new = jnp.where(iou > _THRESH, -1.0, m)
        m_out = jnp.where(vp, m_new, m)

        vec = jnp.where(lane == 0, bx1,
              jnp.where(lane == 1, by1,
              jnp.where(lane == 2, bx2,
              jnp.where(lane == 3, by2,
              jnp.where(lane == 4, gmax, 0.0)))))
        vec = jnp.where(vp, vec, jnp.zeros_like(vec))
        out_ref[0, pl.ds(i, 1), :] = vec
        return m_out

    jax.lax.fori_loop(0, _POST, nms_body, m0)


def kernel(objectness, box_regression, anchors):
    N, A, H, W = objectness.shape
    num = A * H * W
    R = (((num + 127) // 128) + 7) // 8 * 8
    TOT = R * 128

    lg = jnp.transpose(objectness, (0, 2, 3, 1)).reshape(N, num)
    lg = jnp.pad(lg, ((0, 0), (0, TOT - num)),
                 constant_values=-1e9).reshape(N, R, 128)
    br = box_regression.reshape(N, A, 4, H, W).transpose(0, 3, 4, 1, 2)
    br = br.reshape(N, num, 4).transpose(0, 2, 1)
    br = jnp.pad(br, ((0, 0), (0, 0), (0, TOT - num))).reshape(N, 4, R, 128)
    an = anchors.transpose(0, 2, 1)
    an = jnp.pad(an, ((0, 0), (0, 0), (0, TOT - num))).reshape(N, 4, R, 128)

    kern = functools.partial(_rpn_kernel, num, R)
    out = pl.pallas_call(
        kern,
        grid=(N,),
        in_specs=[
            pl.BlockSpec((1, R, 128), lambda b: (b, 0, 0)),
            pl.BlockSpec((1, 4, R, 128), lambda b: (b, 0, 0, 0)),
            pl.BlockSpec((1, 4, R, 128), lambda b: (b, 0, 0, 0)),
        ],
        out_specs=pl.BlockSpec((1, 1024, 128), lambda b: (b, 0, 0)),
        out_shape=jax.ShapeDtypeStruct((N, 1024, 128), jnp.float32),
        scratch_shapes=[pltpu.VMEM((R, 128), jnp.float32)] * 6,
    )(lg, br, an)
    return out[:, :_POST, :5]
